# baseline (speedup 1.0000x reference)
import jax
import jax.numpy as jnp
from jax import lax
from jax.experimental import pallas as pl
from jax.experimental.pallas import tpu as pltpu
from jax.experimental.pallas import tpu_sc as plsc

_OHEM_RATIO = 3.0
_NSTAT = 7
_HWS = (16384, 4096, 1024)
_NIMG = 8
_NCH = 28
_K_SC = 12
_DEPTH = 4


def _f32_from_bits(i):
    return lax.bitcast_convert_type(i, jnp.float32)


def _ce_nll(la, lb, tgt):
    m = jnp.maximum(la, lb)
    lse = m + jnp.log(jnp.exp(la - m) + jnp.exp(lb - m))
    lt = jnp.where(tgt > 0.0, lb, la)
    return lse - lt


def _smooth_l1(d):
    ad = jnp.abs(d)
    return jnp.where(ad < 1.0, 0.5 * d * d, ad - 0.5)


def _sc_wl_body(reg3, gt3, reg4, gt4, reg5, gt5, out, *rest):
    mbufs = rest[0:3]
    pwb = rest[3]
    rbufs = rest[4:4 + _DEPTH]
    wbufs = rest[4 + _DEPTH:4 + 2 * _DEPTH]
    stage = rest[4 + 2 * _DEPTH]
    msem = rest[5 + 2 * _DEPTH]
    rsems = rest[6 + 2 * _DEPTH:6 + 3 * _DEPTH]
    wsems = rest[6 + 3 * _DEPTH:6 + 4 * _DEPTH]

    wid = lax.axis_index("s") * 2 + lax.axis_index("c")
    img = wid // 4
    q = wid % 4
    levels = ((reg3, gt3), (reg4, gt4), (reg5, gt5))

    mh = []
    for lvl, (_, gt_h) in enumerate(levels):
        hw = _HWS[lvl]
        qhw = hw // 4
        hs = []
        for m in range(3):
            hs.append(pltpu.async_copy(
                gt_h.at[pl.ds(img * 31 * hw + m * hw + q * qhw, qhw)],
                mbufs[lvl].at[pl.ds(m * qhw, qhw)], msem))
        mh.append(hs)

    for lvl, (reg_h, gt_h) in enumerate(levels):
        hw = _HWS[lvl]
        qhw = hw // 4
        p0 = q * qhw
        for h in mh[lvl]:
            h.wait()
        mb = mbufs[lvl]

        def _pw(j, carry):
            base = j * 64
            for u in range(4):
                o = base + u * 16
                t = mb[pl.ds(o, 16)]
                tc = mb[pl.ds(qhw + o, 16)]
                tm = mb[pl.ds(2 * qhw + o, 16)]
                pwb[pl.ds(o, 16)] = jnp.where(
                    t * tm > 0.0, (t + tc) * 0.2, 0.0)
            return carry

        lax.fori_loop(0, qhw // 64, _pw, jnp.int32(0))

        def _start(c, slot):
            roff = (img * _NCH + c) * hw + p0
            woff = (img * 31 + 3 + c) * hw + p0
            dst = pl.ds(0, qhw)
            hr = pltpu.async_copy(
                reg_h.at[pl.ds(roff, qhw)], rbufs[slot].at[dst], rsems[slot])
            hwl = pltpu.async_copy(
                gt_h.at[pl.ds(woff, qhw)], wbufs[slot].at[dst], wsems[slot])
            return hr, hwl

        pend = [_start(c, c % _DEPTH) for c in range(min(_DEPTH, _K_SC))]
        accs = (jnp.zeros((16,), jnp.float32),) * 4
        for c in range(_K_SC):
            h2 = pend[c]
            h2[0].wait()
            h2[1].wait()
            if c + _DEPTH < _K_SC:
                pend.append(_start(c + _DEPTH, (c + _DEPTH) % _DEPTH))
            rb = rbufs[c % _DEPTH]
            wb = wbufs[c % _DEPTH]

            def _acc(j, a):
                base = j * 128
                a = list(a)
                for u in range(8):
                    o = base + u * 16
                    sl = pl.ds(o, 16)
                    d = rb[sl] - wb[sl]
                    ad = jnp.abs(d)
                    s = jnp.where(ad < 1.0, 0.5 * d * d, ad - 0.5)
                    a[u % 4] = a[u % 4] + pwb[sl] * s
                return tuple(a)

            accs = lax.fori_loop(0, qhw // 128, _acc, accs)
        stage[...] = accs[0] + accs[1] + accs[2] + accs[3]
        pltpu.sync_copy(stage, out.at[lvl, wid])


def _sc_wl(reg3, gt3, reg4, gt4, reg5, gt5):
    qhw0 = _HWS[0] // 4
    scratch = [pltpu.VMEM((3 * (hw // 4),), jnp.float32) for hw in _HWS]
    scratch.append(pltpu.VMEM((qhw0,), jnp.float32))
    scratch += [pltpu.VMEM((qhw0,), jnp.float32) for _ in range(2 * _DEPTH)]
    scratch.append(pltpu.VMEM((16,), jnp.float32))
    scratch += [pltpu.SemaphoreType.DMA for _ in range(1 + 2 * _DEPTH)]
    f = pl.kernel(
        _sc_wl_body,
        out_type=jax.ShapeDtypeStruct((3, 32, 16), jnp.float32),
        mesh=plsc.VectorSubcoreMesh(core_axis_name="c", subcore_axis_name="s"),
        scratch_types=scratch,
    )
    return f(reg3.reshape(-1), gt3.reshape(-1), reg4.reshape(-1),
             gt4.reshape(-1), reg5.reshape(-1), gt5.reshape(-1))


_NTC = _NCH - _K_SC


def _dense_step(cls_r, gt_r, regs, wls, neg_r, i):
    l0 = cls_r[0, 0]
    l1 = cls_r[0, 1]
    l2 = cls_r[0, 2]
    l3 = cls_r[0, 3]
    tr = gt_r[0, 0]
    tcl = gt_r[0, 1]
    tm = gt_r[0, 2]

    nll_tr = _ce_nll(l0, l1, tr)
    posf = jnp.where(tr * tm > 0.0, 1.0, 0.0).astype(jnp.float32)
    negf = jnp.where((1.0 - tr) * tm > 0.0, 1.0, 0.0).astype(jnp.float32)
    n_pos = jnp.sum(posf)
    n_neg = jnp.sum(negf)
    loss_pos = jnp.sum(posf * nll_tr)
    sum_neg = jnp.sum(negf * nll_tr)
    neg_r[i] = jnp.where(negf > 0.0, nll_tr, -1.0)

    nll_tcl = _ce_nll(l2, l3, tcl)
    s_tcl_pos = jnp.sum(posf * nll_tcl)
    s_tcl_neg = jnp.sum((1.0 - posf) * nll_tcl)

    w = (tr + tcl) * 0.2
    acc = _smooth_l1(regs[0][0, 0] - wls[0][0, 0])
    for c in range(1, _NTC):
        acc = acc + _smooth_l1(regs[c][0, 0] - wls[c][0, 0])
    s_wl = jnp.sum(posf * w * acc)
    return n_pos, n_neg, loss_pos, sum_neg, s_tcl_pos, s_tcl_neg, s_wl


def _topk_sum(neg_r, k, n_neg, sum_neg):
    def _search(_):
        def body(_, lohi):
            lo, hi = lohi
            mid = lo + (hi - lo) // 2
            t = _f32_from_bits(mid)
            cnt = jnp.sum(jnp.where(neg_r[...] >= t, 1.0, 0.0))
            ge = cnt >= k
            return jnp.where(ge, mid, lo), jnp.where(ge, hi, mid)

        lo, _ = lax.fori_loop(
            0, 31, body, (jnp.int32(0), jnp.int32(0x7F800000)))
        v = _f32_from_bits(lo)
        arr = neg_r[...]
        gtm = jnp.where(arr > v, 1.0, 0.0)
        return jnp.sum(arr * gtm) + (k - jnp.sum(gtm)) * v

    return lax.cond(k >= n_neg, lambda _: sum_neg, _search, 0)


_ARGS_PER_LVL = 2 + 2 * _NTC


def _tc_body(*refs):
    nin = 3 * _ARGS_PER_LVL
    out_r = refs[nin]
    negs = refs[nin + 1:nin + 4]
    acc_r = refs[nin + 4]
    i = pl.program_id(0)
    groups = []
    for lvl in range(3):
        o = lvl * _ARGS_PER_LVL
        groups.append((refs[o], refs[o + 1], refs[o + 2:o + 2 + _NTC],
                       refs[o + 2 + _NTC:o + 2 + 2 * _NTC], negs[lvl]))
    for lvl, (cls_r, gt_r, regs, wls, neg_r) in enumerate(groups):
        part = _dense_step(cls_r, gt_r, regs, wls, neg_r, i)
        for j, p in enumerate(part):
            prev = jnp.where(i > 0, acc_r[lvl, j], 0.0)
            acc_r[lvl, j] = prev + p

    @pl.when(i == pl.num_programs(0) - 1)
    def _finalize():
        ltr = jnp.float32(0.0)
        ltcl = jnp.float32(0.0)
        npos = []
        swl = []
        for lvl in range(3):
            neg_r = groups[lvl][4]
            n_pos = acc_r[lvl, 0]
            n_neg = acc_r[lvl, 1]
            loss_pos = acc_r[lvl, 2]
            sum_neg = acc_r[lvl, 3]
            s_tcl_pos = acc_r[lvl, 4]
            s_tcl_neg = acc_r[lvl, 5]
            total = jnp.float32(
                neg_r.shape[0] * neg_r.shape[1] * neg_r.shape[2])
            cap = _OHEM_RATIO * n_pos
            nnb = jnp.minimum(n_neg, cap)
            has = n_pos > 0.0
            k = jnp.where(has, nnb, jnp.minimum(n_neg, 100.0))
            denom = jnp.where(has, n_pos + nnb, 100.0)
            s_top = _topk_sum(neg_r, k, n_neg, sum_neg)
            ltr = ltr + (loss_pos + s_top) / denom
            ltcl = ltcl + jnp.where(
                has, s_tcl_pos / n_pos + 0.5 * s_tcl_neg / (total - n_pos),
                0.0)
            npos.append(n_pos)
            swl.append(acc_r[lvl, 6])

        lane = lax.broadcasted_iota(jnp.int32, (8, 128), 1)
        sub = lax.broadcasted_iota(jnp.int32, (8, 128), 0)
        row0 = sub == 0
        vals = (ltr, ltcl, npos[0], npos[1], npos[2], swl[0], swl[1], swl[2])
        res = jnp.zeros((8, 128), jnp.float32)
        for j, v in enumerate(vals):
            res = res + jnp.where(row0 & (lane == j), v, 0.0)
        out_r[...] = res


def _tc_losses(cls3, reg3, gt3, cls4, reg4, gt4, cls5, reg5, gt5):
    args = []
    in_specs = []
    scratch = []
    n = _NIMG
    for c, r, g in ((cls3, reg3, gt3), (cls4, reg4, gt4), (cls5, reg5, gt5)):
        _, _, h, w = c.shape
        s = (h * w) // 128
        c4 = c.reshape(n, 4, s, 128)
        r4 = r.reshape(n, _NCH, s, 128)
        g4 = g.reshape(n, 31, s, 128)
        args += [c4, g4]
        in_specs.append(pl.BlockSpec((1, 4, s, 128), lambda i: (i, 0, 0, 0)))
        in_specs.append(pl.BlockSpec((1, 4, s, 128), lambda i: (i, 0, 0, 0)))
        for ch in range(_K_SC, _NCH):
            args.append(r4)
            in_specs.append(pl.BlockSpec(
                (1, 1, s, 128), lambda i, ch=ch: (i, ch, 0, 0)))
        for ch in range(_K_SC, _NCH):
            args.append(g4)
            in_specs.append(pl.BlockSpec(
                (1, 1, s, 128), lambda i, ch=ch: (i, 3 + ch, 0, 0)))
        scratch.append(pltpu.VMEM((n, s, 128), jnp.float32))
    scratch.append(pltpu.SMEM((3, _NSTAT), jnp.float32))
    out = pl.pallas_call(
        _tc_body,
        grid=(n,),
        in_specs=in_specs,
        out_specs=pl.BlockSpec((8, 128), lambda i: (0, 0)),
        out_shape=jax.ShapeDtypeStruct((8, 128), jnp.float32),
        scratch_shapes=scratch,
        compiler_params=pltpu.CompilerParams(
            dimension_semantics=("arbitrary",)),
    )(*args)
    return out


def kernel(cls3, reg3, gt3, cls4, reg4, gt4, cls5, reg5, gt5):
    sc_part = _sc_wl(reg3, gt3, reg4, gt4, reg5, gt5)
    tc_out = _tc_losses(cls3, reg3, gt3, cls4, reg4, gt4, cls5, reg5, gt5)
    row = tc_out[0]
    ltr = row[0]
    ltcl = row[1]
    swl_sc = jnp.sum(sc_part, axis=(1, 2))
    lwl = jnp.float32(0.0)
    for lvl in range(3):
        n_pos = row[2 + lvl]
        s_wl = swl_sc[lvl] + row[5 + lvl]
        lwl = lwl + jnp.where(n_pos > 0.0, s_wl / (n_pos * 28.0), 0.0)
    return jnp.stack([ltr, ltcl, lwl])


# speedup vs baseline: 1.0025x; 1.0025x over previous
import jax
import jax.numpy as jnp
from jax import lax
from jax.experimental import pallas as pl
from jax.experimental.pallas import tpu as pltpu
from jax.experimental.pallas import tpu_sc as plsc

_OHEM_RATIO = 3.0
_NSTAT = 7
_HWS = (16384, 4096, 1024)
_NIMG = 8
_NCH = 28
_K_SC = 12
_DEPTH = 4


def _f32_from_bits(i):
    return lax.bitcast_convert_type(i, jnp.float32)


def _ce_nll(la, lb, tgt):
    m = jnp.maximum(la, lb)
    lse = m + jnp.log(jnp.exp(la - m) + jnp.exp(lb - m))
    lt = jnp.where(tgt > 0.0, lb, la)
    return lse - lt


def _smooth_l1(d):
    ad = jnp.abs(d)
    return jnp.where(ad < 1.0, 0.5 * d * d, ad - 0.5)


def _sc_wl_body(reg3, gt3, reg4, gt4, reg5, gt5, out, *rest):
    mbufs = rest[0:3]
    pwb = rest[3]
    rbufs = rest[4:4 + _DEPTH]
    wbufs = rest[4 + _DEPTH:4 + 2 * _DEPTH]
    stage = rest[4 + 2 * _DEPTH]
    msem = rest[5 + 2 * _DEPTH]
    rsems = rest[6 + 2 * _DEPTH:6 + 3 * _DEPTH]
    wsems = rest[6 + 3 * _DEPTH:6 + 4 * _DEPTH]

    wid = lax.axis_index("s") * 2 + lax.axis_index("c")
    img = wid // 4
    q = wid % 4
    levels = ((reg3, gt3), (reg4, gt4), (reg5, gt5))

    mh = []
    for lvl, (_, gt_h) in enumerate(levels):
        hw = _HWS[lvl]
        qhw = hw // 4
        hs = []
        for m in range(3):
            hs.append(pltpu.async_copy(
                gt_h.at[pl.ds(img * 31 * hw + m * hw + q * qhw, qhw)],
                mbufs[lvl].at[pl.ds(m * qhw, qhw)], msem))
        mh.append(hs)

    for lvl, (reg_h, gt_h) in enumerate(levels):
        hw = _HWS[lvl]
        qhw = hw // 4
        p0 = q * qhw
        for h in mh[lvl]:
            h.wait()
        mb = mbufs[lvl]

        def _pw(j, carry):
            base = j * 64
            for u in range(4):
                o = base + u * 16
                t = mb[pl.ds(o, 16)]
                tc = mb[pl.ds(qhw + o, 16)]
                tm = mb[pl.ds(2 * qhw + o, 16)]
                pwb[pl.ds(o, 16)] = jnp.where(
                    t * tm > 0.0, (t + tc) * 0.2, 0.0)
            return carry

        lax.fori_loop(0, qhw // 64, _pw, jnp.int32(0))

        def _start(c, slot):
            roff = (img * _NCH + c) * hw + p0
            woff = (img * 31 + 3 + c) * hw + p0
            dst = pl.ds(0, qhw)
            hr = pltpu.async_copy(
                reg_h.at[pl.ds(roff, qhw)], rbufs[slot].at[dst], rsems[slot])
            hwl = pltpu.async_copy(
                gt_h.at[pl.ds(woff, qhw)], wbufs[slot].at[dst], wsems[slot])
            return hr, hwl

        pend = [_start(c, c % _DEPTH) for c in range(min(_DEPTH, _K_SC))]
        accs = (jnp.zeros((16,), jnp.float32),) * 4
        for c in range(_K_SC):
            h2 = pend[c]
            h2[0].wait()
            h2[1].wait()
            if c + _DEPTH < _K_SC:
                pend.append(_start(c + _DEPTH, (c + _DEPTH) % _DEPTH))
            rb = rbufs[c % _DEPTH]
            wb = wbufs[c % _DEPTH]

            def _acc(j, a):
                base = j * 128
                a = list(a)
                for u in range(8):
                    o = base + u * 16
                    sl = pl.ds(o, 16)
                    d = rb[sl] - wb[sl]
                    ad = jnp.abs(d)
                    s = jnp.where(ad < 1.0, 0.5 * d * d, ad - 0.5)
                    a[u % 4] = a[u % 4] + pwb[sl] * s
                return tuple(a)

            accs = lax.fori_loop(0, qhw // 128, _acc, accs)
        stage[...] = accs[0] + accs[1] + accs[2] + accs[3]
        pltpu.sync_copy(stage, out.at[lvl, wid])


def _sc_wl(reg3, gt3, reg4, gt4, reg5, gt5):
    qhw0 = _HWS[0] // 4
    scratch = [pltpu.VMEM((3 * (hw // 4),), jnp.float32) for hw in _HWS]
    scratch.append(pltpu.VMEM((qhw0,), jnp.float32))
    scratch += [pltpu.VMEM((qhw0,), jnp.float32) for _ in range(2 * _DEPTH)]
    scratch.append(pltpu.VMEM((16,), jnp.float32))
    scratch += [pltpu.SemaphoreType.DMA for _ in range(1 + 2 * _DEPTH)]
    f = pl.kernel(
        _sc_wl_body,
        out_type=jax.ShapeDtypeStruct((3, 32, 16), jnp.float32),
        mesh=plsc.VectorSubcoreMesh(core_axis_name="c", subcore_axis_name="s"),
        scratch_types=scratch,
    )
    return f(reg3.reshape(-1), gt3.reshape(-1), reg4.reshape(-1),
             gt4.reshape(-1), reg5.reshape(-1), gt5.reshape(-1))


_NTC = _NCH - _K_SC


def _dense_step(cls_r, gt_r, regs, wls, neg_r, i):
    l0 = cls_r[0, 0]
    l1 = cls_r[0, 1]
    l2 = cls_r[0, 2]
    l3 = cls_r[0, 3]
    tr = gt_r[0, 0]
    tcl = gt_r[0, 1]
    tm = gt_r[0, 2]

    nll_tr = _ce_nll(l0, l1, tr)
    posf = jnp.where(tr * tm > 0.0, 1.0, 0.0).astype(jnp.float32)
    negf = jnp.where((1.0 - tr) * tm > 0.0, 1.0, 0.0).astype(jnp.float32)
    n_pos = jnp.sum(posf)
    n_neg = jnp.sum(negf)
    loss_pos = jnp.sum(posf * nll_tr)
    sum_neg = jnp.sum(negf * nll_tr)
    neg_r[i] = jnp.where(negf > 0.0, nll_tr, -1.0)

    nll_tcl = _ce_nll(l2, l3, tcl)
    s_tcl_pos = jnp.sum(posf * nll_tcl)
    s_tcl_neg = jnp.sum((1.0 - posf) * nll_tcl)

    w = (tr + tcl) * 0.2
    rchans = [rb[0, j] for rb in regs for j in range(rb.shape[1])]
    wchans = [wb[0, j] for wb in wls for j in range(wb.shape[1])]
    acc = _smooth_l1(rchans[0] - wchans[0])
    for c in range(1, _NTC):
        acc = acc + _smooth_l1(rchans[c] - wchans[c])
    s_wl = jnp.sum(posf * w * acc)
    return n_pos, n_neg, loss_pos, sum_neg, s_tcl_pos, s_tcl_neg, s_wl


def _topk_sum(neg_r, k, n_neg, sum_neg):
    def _search(_):
        def body(_, lohi):
            lo, hi = lohi
            mid = lo + (hi - lo) // 2
            t = _f32_from_bits(mid)
            cnt = jnp.sum(jnp.where(neg_r[...] >= t, 1.0, 0.0))
            ge = cnt >= k
            return jnp.where(ge, mid, lo), jnp.where(ge, hi, mid)

        lo, _ = lax.fori_loop(
            0, 31, body, (jnp.int32(0), jnp.int32(0x7F800000)))
        v = _f32_from_bits(lo)
        arr = neg_r[...]
        gtm = jnp.where(arr > v, 1.0, 0.0)
        return jnp.sum(arr * gtm) + (k - jnp.sum(gtm)) * v

    return lax.cond(k >= n_neg, lambda _: sum_neg, _search, 0)


_ARGS_PER_LVL = 2 + 4 + 4


def _tc_body(*refs):
    nin = 3 * _ARGS_PER_LVL
    out_r = refs[nin]
    negs = refs[nin + 1:nin + 4]
    acc_r = refs[nin + 4]
    i = pl.program_id(0)
    groups = []
    for lvl in range(3):
        o = lvl * _ARGS_PER_LVL
        groups.append((refs[o], refs[o + 1], refs[o + 2:o + 6],
                       refs[o + 6:o + 10], negs[lvl]))
    for lvl, (cls_r, gt_r, regs, wls, neg_r) in enumerate(groups):
        part = _dense_step(cls_r, gt_r, regs, wls, neg_r, i)
        for j, p in enumerate(part):
            prev = jnp.where(i > 0, acc_r[lvl, j], 0.0)
            acc_r[lvl, j] = prev + p

    @pl.when(i == pl.num_programs(0) - 1)
    def _finalize():
        ltr = jnp.float32(0.0)
        ltcl = jnp.float32(0.0)
        npos = []
        swl = []
        for lvl in range(3):
            neg_r = groups[lvl][4]
            n_pos = acc_r[lvl, 0]
            n_neg = acc_r[lvl, 1]
            loss_pos = acc_r[lvl, 2]
            sum_neg = acc_r[lvl, 3]
            s_tcl_pos = acc_r[lvl, 4]
            s_tcl_neg = acc_r[lvl, 5]
            total = jnp.float32(
                neg_r.shape[0] * neg_r.shape[1] * neg_r.shape[2])
            cap = _OHEM_RATIO * n_pos
            nnb = jnp.minimum(n_neg, cap)
            has = n_pos > 0.0
            k = jnp.where(has, nnb, jnp.minimum(n_neg, 100.0))
            denom = jnp.where(has, n_pos + nnb, 100.0)
            s_top = _topk_sum(neg_r, k, n_neg, sum_neg)
            ltr = ltr + (loss_pos + s_top) / denom
            ltcl = ltcl + jnp.where(
                has, s_tcl_pos / n_pos + 0.5 * s_tcl_neg / (total - n_pos),
                0.0)
            npos.append(n_pos)
            swl.append(acc_r[lvl, 6])

        lane = lax.broadcasted_iota(jnp.int32, (8, 128), 1)
        sub = lax.broadcasted_iota(jnp.int32, (8, 128), 0)
        row0 = sub == 0
        vals = (ltr, ltcl, npos[0], npos[1], npos[2], swl[0], swl[1], swl[2])
        res = jnp.zeros((8, 128), jnp.float32)
        for j, v in enumerate(vals):
            res = res + jnp.where(row0 & (lane == j), v, 0.0)
        out_r[...] = res


def _tc_losses(cls3, reg3, gt3, cls4, reg4, gt4, cls5, reg5, gt5):
    args = []
    in_specs = []
    scratch = []
    n = _NIMG
    for c, r, g in ((cls3, reg3, gt3), (cls4, reg4, gt4), (cls5, reg5, gt5)):
        _, _, h, w = c.shape
        s = (h * w) // 128
        c4 = c.reshape(n, 4, s, 128)
        r4 = r.reshape(n, _NCH, s, 128)
        g4 = g.reshape(n, 31, s, 128)
        args += [c4, g4]
        in_specs.append(pl.BlockSpec((1, 4, s, 128), lambda i: (i, 0, 0, 0)))
        in_specs.append(pl.BlockSpec((1, 4, s, 128), lambda i: (i, 0, 0, 0)))
        for bi in (3, 4, 5, 6):
            args.append(r4)
            in_specs.append(pl.BlockSpec(
                (1, 4, s, 128), lambda i, bi=bi: (i, bi, 0, 0)))
        for bi in (3, 4, 5):
            args.append(g4)
            in_specs.append(pl.BlockSpec(
                (1, 5, s, 128), lambda i, bi=bi: (i, bi, 0, 0)))
        args.append(g4)
        in_specs.append(pl.BlockSpec((1, 1, s, 128), lambda i: (i, 30, 0, 0)))
        scratch.append(pltpu.VMEM((n, s, 128), jnp.float32))
    scratch.append(pltpu.SMEM((3, _NSTAT), jnp.float32))
    out = pl.pallas_call(
        _tc_body,
        grid=(n,),
        in_specs=in_specs,
        out_specs=pl.BlockSpec((8, 128), lambda i: (0, 0)),
        out_shape=jax.ShapeDtypeStruct((8, 128), jnp.float32),
        scratch_shapes=scratch,
        compiler_params=pltpu.CompilerParams(
            dimension_semantics=("arbitrary",)),
    )(*args)
    return out


def kernel(cls3, reg3, gt3, cls4, reg4, gt4, cls5, reg5, gt5):
    sc_part = _sc_wl(reg3, gt3, reg4, gt4, reg5, gt5)
    tc_out = _tc_losses(cls3, reg3, gt3, cls4, reg4, gt4, cls5, reg5, gt5)
    row = tc_out[0]
    ltr = row[0]
    ltcl = row[1]
    swl_sc = jnp.sum(sc_part, axis=(1, 2))
    lwl = jnp.float32(0.0)
    for lvl in range(3):
        n_pos = row[2 + lvl]
        s_wl = swl_sc[lvl] + row[5 + lvl]
        lwl = lwl + jnp.where(n_pos > 0.0, s_wl / (n_pos * 28.0), 0.0)
    return jnp.stack([ltr, ltcl, lwl])
